# Initial kernel scaffold; baseline (speedup 1.0000x reference)
import jax
import jax.numpy as jnp
from jax.experimental import pallas as pl


def kernel(input, batch):
    raise NotImplementedError("write your pallas kernel here")


# speedup vs baseline: 4.0742x; 4.0742x over previous
import jax
import jax.numpy as jnp
from jax.experimental import pallas as pl
from jax.experimental.pallas import tpu as pltpu


_B = 256


def _main_body(nb, x_ref, ids_ref, gram_ref, sc_ref, bs_ref):
    i = pl.program_id(0)
    X = x_ref[...]
    g = jax.lax.dot_general(X, X, (((0,), (0,)), ((), ())),
                            preferred_element_type=jnp.float32)
    norms = jnp.sqrt(jnp.sum(X * X, axis=1, keepdims=True))
    ids = ids_ref[0, :, :]
    onehot_t = (jax.lax.broadcasted_iota(jnp.int32, (_B, nb), 0) == ids
                ).astype(jnp.float32)
    rhs = jnp.concatenate([norms, jnp.ones_like(norms)], axis=1)
    sc = jax.lax.dot_general(onehot_t, rhs, (((1,), (0,)), ((), ())),
                             preferred_element_type=jnp.float32)
    bmax = jnp.max(ids)

    @pl.when(i == 0)
    def _init():
        gram_ref[...] = g
        sc_ref[...] = sc
        bs_ref[0, 0] = bmax

    @pl.when(i > 0)
    def _acc():
        gram_ref[...] += g
        sc_ref[...] += sc
        bs_ref[0, 0] = jnp.maximum(bs_ref[0, 0], bmax)


def _epilogue_body(g_ref, sc_ref, bs_ref, nm_ref, sr_ref):
    G0 = g_ref[...]
    row = jax.lax.broadcasted_iota(jnp.int32, (128, 128), 0)
    col = jax.lax.broadcasted_iota(jnp.int32, (128, 128), 1)
    fro2 = jnp.sum(jnp.where(row == col, G0, 0.0))

    Gm = G0
    for _ in range(20):
        s = jnp.max(jnp.abs(Gm))
        Gn = Gm / jnp.maximum(s, 1e-30)
        Gm = jax.lax.dot_general(Gn, Gn, (((1,), (0,)), ((), ())),
                                 preferred_element_type=jnp.float32)
    coln = jnp.sum(Gm * Gm, axis=0, keepdims=True)
    lane = jax.lax.broadcasted_iota(jnp.int32, (1, 128), 1)
    j = jnp.min(jnp.where(coln == jnp.max(coln), lane, 256))
    v_col = jnp.sum(jnp.where(col == j, Gm, 0.0), axis=1, keepdims=True)
    v_row = jnp.sum(jnp.where(row == j, Gm, 0.0), axis=0, keepdims=True)
    t_col = jnp.sum(G0 * v_row, axis=1, keepdims=True)
    sigma_max_sq = jnp.sum(t_col * v_col) / jnp.sum(v_col * v_col)
    sr_ref[0, 0] = fro2 / sigma_max_sq

    seg = sc_ref[:, 0:1]
    cnt = sc_ref[:, 1:2]
    per_graph = seg / jnp.maximum(cnt, 1.0)
    bidx = jax.lax.broadcasted_iota(jnp.int32, (_B, 1), 0)
    bs = bs_ref[0, 0]
    nm = jnp.sum(jnp.where(bidx < bs, per_graph, 0.0))
    nm_ref[0, 0] = nm / (bs + 1).astype(jnp.float32)


def kernel(input, batch):
    n, d = input.shape
    assert d == 128
    nb = 6400 if n % 6400 == 0 else n
    grid = n // nb
    ids3 = batch.astype(jnp.int32).reshape(grid, 1, nb)

    gram, sc, bs = pl.pallas_call(
        lambda *refs: _main_body(nb, *refs),
        grid=(grid,),
        in_specs=[
            pl.BlockSpec((nb, d), lambda i: (i, 0)),
            pl.BlockSpec((1, 1, nb), lambda i: (i, 0, 0)),
        ],
        out_specs=[
            pl.BlockSpec((d, d), lambda i: (0, 0)),
            pl.BlockSpec((_B, 2), lambda i: (0, 0)),
            pl.BlockSpec((1, 1), lambda i: (0, 0),
                         memory_space=pltpu.SMEM),
        ],
        out_shape=[
            jax.ShapeDtypeStruct((d, d), jnp.float32),
            jax.ShapeDtypeStruct((_B, 2), jnp.float32),
            jax.ShapeDtypeStruct((1, 1), jnp.int32),
        ],
    )(input, ids3)

    nm, sr = pl.pallas_call(
        _epilogue_body,
        in_specs=[
            pl.BlockSpec((d, d), lambda: (0, 0)),
            pl.BlockSpec((_B, 2), lambda: (0, 0)),
            pl.BlockSpec((1, 1), lambda: (0, 0), memory_space=pltpu.SMEM),
        ],
        out_specs=[
            pl.BlockSpec((1, 1), lambda: (0, 0), memory_space=pltpu.SMEM),
            pl.BlockSpec((1, 1), lambda: (0, 0), memory_space=pltpu.SMEM),
        ],
        out_shape=[
            jax.ShapeDtypeStruct((1, 1), jnp.float32),
            jax.ShapeDtypeStruct((1, 1), jnp.float32),
        ],
    )(gram, sc, bs)

    return (input, nm[0, 0], sr[0, 0])
